# Initial kernel scaffold; baseline (speedup 1.0000x reference)
import jax
import jax.numpy as jnp
from jax.experimental import pallas as pl


def kernel(indices, symbol_counts):
    raise NotImplementedError("write your pallas kernel here")


# speedup vs baseline: 218.1276x; 218.1276x over previous
import functools

import jax
import jax.numpy as jnp
from jax import lax
from jax.experimental import pallas as pl
from jax.experimental.pallas import tpu as pltpu
from jax.experimental.pallas import tpu_sc as plsc

_CODEBOOK = 100000
_B, _T = 16384, 200
_N = _B * _T

_NC, _NS, _L = 2, 16, 16
_NW = _NC * _NS
_PER_TILE = _N // _NW
_CH = 12800
_NCHUNK = _PER_TILE // _CH


@functools.lru_cache(maxsize=1)
def _make_hist_kernel():
    mesh = plsc.VectorSubcoreMesh(core_axis_name="c", subcore_axis_name="s")

    @functools.partial(
        pl.kernel,
        mesh=mesh,
        out_type=jax.ShapeDtypeStruct((_NW, _CODEBOOK), jnp.int32),
        scratch_types=[
            pltpu.VMEM((_CODEBOOK,), jnp.int32),
            pltpu.VMEM((_CH,), jnp.int32),
        ],
        compiler_params=pltpu.CompilerParams(needs_layout_passes=False),
    )
    def hist_kernel(idx_hbm, out_hbm, hist_v, buf_v):
        wid = lax.axis_index("s") * _NC + lax.axis_index("c")
        base = wid * _PER_TILE
        ones = jnp.full((_L,), 1, dtype=jnp.int32)
        zeros = jnp.zeros((_L,), dtype=jnp.int32)

        def zero_body(i, carry):
            hist_v[pl.ds(i * _L, _L)] = zeros
            return carry

        lax.fori_loop(0, _CODEBOOK // _L, zero_body, 0)

        def chunk_body(c, carry):
            pltpu.sync_copy(idx_hbm.at[pl.ds(base + c * _CH, _CH)], buf_v)

            def group_body(g, carry2):
                idx = buf_v[pl.ds(g * _L, _L)]
                plsc.addupdate_scatter(hist_v, [idx], ones)
                return carry2

            lax.fori_loop(0, _CH // _L, group_body, 0)
            return carry

        lax.fori_loop(0, _NCHUNK, chunk_body, 0)
        pltpu.sync_copy(hist_v, out_hbm.at[wid])

    return hist_kernel


def _finalize_body(parts_ref, counts_ref, out_ref):
    hist_i = jnp.sum(parts_ref[...], axis=0, keepdims=True)
    hist = hist_i.astype(jnp.float32)
    smoothed = counts_ref[...] + hist + 1e-8
    total = jnp.sum(smoothed)
    probs = smoothed / total
    bits = jnp.log(jnp.maximum(probs, 1e-10)) * (-1.4426950408889634)
    out_ref[0, 0] = jnp.sum(hist * bits) / _N


def _finalize(parts, counts2d):
    return pl.pallas_call(
        _finalize_body,
        out_shape=jax.ShapeDtypeStruct((1, 1), jnp.float32),
        in_specs=[
            pl.BlockSpec(memory_space=pltpu.VMEM),
            pl.BlockSpec(memory_space=pltpu.VMEM),
        ],
        out_specs=pl.BlockSpec(memory_space=pltpu.SMEM),
    )(parts, counts2d)


def kernel(indices, symbol_counts):
    flat = indices.reshape(-1)
    parts = _make_hist_kernel()(flat)
    out = _finalize(parts, symbol_counts.reshape(1, _CODEBOOK))
    return out.reshape(())


# speedup vs baseline: 291.0087x; 1.3341x over previous
import functools

import jax
import jax.numpy as jnp
from jax import lax
from jax.experimental import pallas as pl
from jax.experimental.pallas import tpu as pltpu
from jax.experimental.pallas import tpu_sc as plsc

_CODEBOOK = 100000
_B, _T = 16384, 200
_N = _B * _T

_NC, _NS, _L = 2, 16, 16
_NW = _NC * _NS
_PER_TILE = _N // _NW
_CH = 12800
_NCHUNK = _PER_TILE // _CH


@functools.lru_cache(maxsize=1)
def _make_hist_kernel():
    mesh = plsc.VectorSubcoreMesh(core_axis_name="c", subcore_axis_name="s")

    @functools.partial(
        pl.kernel,
        mesh=mesh,
        out_type=jax.ShapeDtypeStruct((_NW, _CODEBOOK), jnp.int32),
        scratch_types=[
            pltpu.VMEM((_CODEBOOK,), jnp.int32),
            pltpu.VMEM((_CH,), jnp.int32),
            pltpu.VMEM((_CH,), jnp.int32),
            pltpu.SemaphoreType.DMA,
            pltpu.SemaphoreType.DMA,
        ],
        compiler_params=pltpu.CompilerParams(needs_layout_passes=False),
    )
    def hist_kernel(idx_hbm, out_hbm, hist_v, buf0_v, buf1_v, sem0, sem1):
        wid = lax.axis_index("s") * _NC + lax.axis_index("c")
        base = wid * _PER_TILE
        ones = jnp.full((_L,), 1, dtype=jnp.int32)
        zeros = jnp.zeros((_L,), dtype=jnp.int32)
        bufs = (buf0_v, buf1_v)
        sems = (sem0, sem1)

        def start(c):
            return pltpu.async_copy(
                idx_hbm.at[pl.ds(base + c * _CH, _CH)], bufs[c % 2], sems[c % 2]
            )

        cps = {0: start(0), 1: start(1)}

        _ZU = 10
        def zero_body(i, carry):
            for k in range(_ZU):
                hist_v[pl.ds((i * _ZU + k) * _L, _L)] = zeros
            return carry

        lax.fori_loop(0, _CODEBOOK // _L // _ZU, zero_body, 0)

        _GU = 8
        for c in range(_NCHUNK):
            cps[c].wait()
            buf = bufs[c % 2]

            def group_body(g, carry, buf=buf):
                for k in range(_GU):
                    idx = buf[pl.ds((g * _GU + k) * _L, _L)]
                    plsc.addupdate_scatter(hist_v, [idx], ones)
                return carry

            lax.fori_loop(0, _CH // _L // _GU, group_body, 0)
            if c + 2 < _NCHUNK:
                cps[c + 2] = start(c + 2)

        pltpu.sync_copy(hist_v, out_hbm.at[wid])

    return hist_kernel


def _finalize_body(parts_ref, counts_ref, out_ref):
    hist_i = jnp.sum(parts_ref[...], axis=0, keepdims=True)
    hist = hist_i.astype(jnp.float32)
    smoothed = counts_ref[...] + hist + 1e-8
    total = jnp.sum(smoothed)
    probs = smoothed / total
    bits = jnp.log(jnp.maximum(probs, 1e-10)) * (-1.4426950408889634)
    out_ref[0, 0] = jnp.sum(hist * bits) / _N


def _finalize(parts, counts2d):
    return pl.pallas_call(
        _finalize_body,
        out_shape=jax.ShapeDtypeStruct((1, 1), jnp.float32),
        in_specs=[
            pl.BlockSpec(memory_space=pltpu.VMEM),
            pl.BlockSpec(memory_space=pltpu.VMEM),
        ],
        out_specs=pl.BlockSpec(memory_space=pltpu.SMEM),
    )(parts, counts2d)


def kernel(indices, symbol_counts):
    flat = indices.reshape(-1)
    parts = _make_hist_kernel()(flat)
    out = _finalize(parts, symbol_counts.reshape(1, _CODEBOOK))
    return out.reshape(())


# speedup vs baseline: 370.5414x; 1.2733x over previous
import functools

import jax
import jax.numpy as jnp
from jax import lax
from jax.experimental import pallas as pl
from jax.experimental.pallas import tpu as pltpu
from jax.experimental.pallas import tpu_sc as plsc

_CODEBOOK = 100000
_B, _T = 16384, 200
_N = _B * _T

_NC, _NS, _L = 2, 16, 16
_NW = _NC * _NS
_ROWS_PER_TILE = _B // _NW
_CHR = 32
_NCHUNK = _ROWS_PER_TILE // _CHR
_FULL_GROUPS = _T // _L
_TAIL = _T - _FULL_GROUPS * _L


@functools.lru_cache(maxsize=1)
def _make_hist_kernel():
    mesh = plsc.VectorSubcoreMesh(core_axis_name="c", subcore_axis_name="s")

    @functools.partial(
        pl.kernel,
        mesh=mesh,
        out_type=jax.ShapeDtypeStruct((_NW, _CODEBOOK), jnp.int32),
        scratch_types=[
            pltpu.VMEM((_CODEBOOK,), jnp.int32),
            pltpu.VMEM((_CHR, _T), jnp.int32),
            pltpu.VMEM((_CHR, _T), jnp.int32),
            pltpu.SemaphoreType.DMA,
            pltpu.SemaphoreType.DMA,
        ],
        compiler_params=pltpu.CompilerParams(needs_layout_passes=False),
    )
    def hist_kernel(idx_hbm, out_hbm, hist_v, buf0_v, buf1_v, sem0, sem1):
        wid = lax.axis_index("s") * _NC + lax.axis_index("c")
        row_base = wid * _ROWS_PER_TILE
        ones = jnp.full((_L,), 1, dtype=jnp.int32)
        zeros = jnp.zeros((_L,), dtype=jnp.int32)
        tail_mask = lax.iota(jnp.int32, _L) >= (_L - _TAIL)
        bufs = (buf0_v, buf1_v)
        sems = (sem0, sem1)

        def start(c):
            return pltpu.async_copy(
                idx_hbm.at[pl.ds(row_base + c * _CHR, _CHR)],
                bufs[c % 2],
                sems[c % 2],
            )

        cps = {0: start(0), 1: start(1)}

        _ZU = 10
        def zero_body(i, carry):
            for k in range(_ZU):
                hist_v[pl.ds((i * _ZU + k) * _L, _L)] = zeros
            return carry

        lax.fori_loop(0, _CODEBOOK // _L // _ZU, zero_body, 0)

        for c in range(_NCHUNK):
            cps[c].wait()
            buf = bufs[c % 2]

            def row_body(r, carry, buf=buf):
                for g in range(_FULL_GROUPS):
                    idx = buf[r, pl.ds(g * _L, _L)]
                    plsc.addupdate_scatter(hist_v, [idx], ones)
                idx = buf[r, pl.ds(_T - _L, _L)]
                plsc.addupdate_scatter(hist_v, [idx], ones, mask=tail_mask)
                return carry

            lax.fori_loop(0, _CHR, row_body, 0)
            if c + 2 < _NCHUNK:
                cps[c + 2] = start(c + 2)

        pltpu.sync_copy(hist_v, out_hbm.at[wid])

    return hist_kernel


def _finalize_body(parts_ref, counts_ref, out_ref):
    hist_i = jnp.sum(parts_ref[...], axis=0, keepdims=True)
    hist = hist_i.astype(jnp.float32)
    smoothed = counts_ref[...] + hist + 1e-8
    total = jnp.sum(smoothed)
    probs = smoothed / total
    bits = jnp.log(jnp.maximum(probs, 1e-10)) * (-1.4426950408889634)
    out_ref[0, 0] = jnp.sum(hist * bits) / _N


def _finalize(parts, counts2d):
    return pl.pallas_call(
        _finalize_body,
        out_shape=jax.ShapeDtypeStruct((1, 1), jnp.float32),
        in_specs=[
            pl.BlockSpec(memory_space=pltpu.VMEM),
            pl.BlockSpec(memory_space=pltpu.VMEM),
        ],
        out_specs=pl.BlockSpec(memory_space=pltpu.SMEM),
    )(parts, counts2d)


def kernel(indices, symbol_counts):
    parts = _make_hist_kernel()(indices)
    out = _finalize(parts, symbol_counts.reshape(1, _CODEBOOK))
    return out.reshape(())


# speedup vs baseline: 512.5154x; 1.3832x over previous
import functools

import jax
import jax.numpy as jnp
from jax import lax
from jax.experimental import pallas as pl
from jax.experimental.pallas import tpu as pltpu
from jax.experimental.pallas import tpu_sc as plsc

_CODEBOOK = 100000
_B, _T = 16384, 200
_N = _B * _T

_NC, _NS, _L = 2, 16, 16
_NW = _NC * _NS
_ROWS_PER_TILE = _B // _NW
_CHR = 32
_NCHUNK = _ROWS_PER_TILE // _CHR
_FULL_GROUPS = _T // _L
_TAIL = _T - _FULL_GROUPS * _L


@functools.lru_cache(maxsize=1)
def _make_hist_kernel():
    mesh = plsc.VectorSubcoreMesh(core_axis_name="c", subcore_axis_name="s")

    @functools.partial(
        pl.kernel,
        mesh=mesh,
        out_type=jax.ShapeDtypeStruct((_NW, _CODEBOOK), jnp.int32),
        scratch_types=[
            pltpu.VMEM((_CODEBOOK,), jnp.int32),
            pltpu.VMEM((_CHR, _T), jnp.int32),
            pltpu.VMEM((_CHR, _T), jnp.int32),
            pltpu.SemaphoreType.DMA,
            pltpu.SemaphoreType.DMA,
        ],
        compiler_params=pltpu.CompilerParams(needs_layout_passes=False),
    )
    def hist_kernel(idx_hbm, out_hbm, hist_v, buf0_v, buf1_v, sem0, sem1):
        wid = lax.axis_index("s") * _NC + lax.axis_index("c")
        row_base = wid * _ROWS_PER_TILE
        ones = jnp.full((_L,), 1, dtype=jnp.int32)
        zeros = jnp.zeros((_L,), dtype=jnp.int32)
        tail_mask = lax.iota(jnp.int32, _L) >= (_L - _TAIL)
        bufs = (buf0_v, buf1_v)
        sems = (sem0, sem1)

        def start(c):
            return pltpu.async_copy(
                idx_hbm.at[pl.ds(row_base + c * _CHR, _CHR)],
                bufs[c % 2],
                sems[c % 2],
            )

        cps = {0: start(0), 1: start(1)}

        @plsc.parallel_loop(0, _CODEBOOK // _L, unroll=8)
        def _zero(i):
            hist_v[pl.ds(i * _L, _L)] = zeros

        for c in range(_NCHUNK):
            cps[c].wait()
            buf = bufs[c % 2]

            def row_body(r, buf=buf):
                for g in range(_FULL_GROUPS):
                    idx = buf[r, pl.ds(g * _L, _L)]
                    plsc.addupdate_scatter(hist_v, [idx], ones)
                idx = buf[r, pl.ds(_T - _L, _L)]
                plsc.addupdate_scatter(hist_v, [idx], ones, mask=tail_mask)

            plsc.parallel_loop(0, _CHR, unroll=2)(row_body)
            if c + 2 < _NCHUNK:
                cps[c + 2] = start(c + 2)

        pltpu.sync_copy(hist_v, out_hbm.at[wid])

    return hist_kernel


def _finalize_body(parts_ref, counts_ref, out_ref):
    hist_i = jnp.sum(parts_ref[...], axis=0, keepdims=True)
    hist = hist_i.astype(jnp.float32)
    smoothed = counts_ref[...] + hist + 1e-8
    total = jnp.sum(smoothed)
    probs = smoothed / total
    bits = jnp.log(jnp.maximum(probs, 1e-10)) * (-1.4426950408889634)
    out_ref[0, 0] = jnp.sum(hist * bits) / _N


def _finalize(parts, counts2d):
    return pl.pallas_call(
        _finalize_body,
        out_shape=jax.ShapeDtypeStruct((1, 1), jnp.float32),
        in_specs=[
            pl.BlockSpec(memory_space=pltpu.VMEM),
            pl.BlockSpec(memory_space=pltpu.VMEM),
        ],
        out_specs=pl.BlockSpec(memory_space=pltpu.SMEM),
    )(parts, counts2d)


def kernel(indices, symbol_counts):
    parts = _make_hist_kernel()(indices)
    out = _finalize(parts, symbol_counts.reshape(1, _CODEBOOK))
    return out.reshape(())
